# baseline (speedup 1.0000x reference)
import dataclasses
import functools

import jax
import jax.numpy as jnp
import numpy as np
from jax import lax
from jax.experimental import pallas as pl
from jax.experimental.pallas import tpu as pltpu
from jax.experimental.pallas import tpu_sc as plsc

N = 10000
E = 320000
D = 128
DH = D // 2
DE = 16
NC, NS, L = 2, 16, 16
NW = NC * NS
EPW = E // NW
CH = 80
NCHUNK = EPW // CH
NST = 10
RPT = N // NST
G = D // (2 * L)

_PERM = np.concatenate(
    [np.concatenate([16 * k + np.arange(16), 64 + 16 * k + np.arange(16)])
     for k in range(G)]
)

_mesh = plsc.VectorSubcoreMesh(
    core_axis_name="c", subcore_axis_name="s", num_cores=NC, num_subcores=NS
)

_sc_params = pltpu.CompilerParams()
if "needs_layout_passes" in pltpu.CompilerParams.__dataclass_fields__:
    _sc_params = dataclasses.replace(_sc_params, needs_layout_passes=False)
if "use_tc_tiling_on_sc" in pltpu.CompilerParams.__dataclass_fields__:
    _sc_params = dataclasses.replace(_sc_params, use_tc_tiling_on_sc=False)


def _pack_pairs(v):
    lo = v[:, :DH].astype(jnp.bfloat16).astype(jnp.float32)
    hi = v[:, DH:].astype(jnp.bfloat16).astype(jnp.float32)
    ulo = lax.shift_right_logical(lax.bitcast_convert_type(lo, jnp.uint32),
                                  jnp.uint32(16))
    uhi = lax.bitwise_and(lax.bitcast_convert_type(hi, jnp.uint32),
                          jnp.uint32(0xFFFF0000))
    return lax.bitcast_convert_type(lax.bitwise_or(ulo, uhi), jnp.float32)


def _node_proj_body(x_ref, w_ref, xa_ref, xb_ref):
    xab = jnp.dot(x_ref[...], w_ref[...], preferred_element_type=jnp.float32)
    xa_ref[...] = _pack_pairs(xab[:, :D])
    xb_ref[...] = _pack_pairs(xab[:, D:])


def _node_proj(x, w1abt):
    return pl.pallas_call(
        _node_proj_body,
        out_shape=(
            jax.ShapeDtypeStruct((N, DH), jnp.float32),
            jax.ShapeDtypeStruct((N, DH), jnp.float32),
        ),
    )(x, w1abt)


def _edge_proj_body(ea_ref, w_ref, b_ref, out_ref):
    v = (jnp.dot(ea_ref[...], w_ref[...], preferred_element_type=jnp.float32)
         + b_ref[...])
    out_ref[...] = _pack_pairs(v)


def _edge_proj(edge_attr, w1ct, b1):
    BE = 10000
    return pl.pallas_call(
        _edge_proj_body,
        out_shape=jax.ShapeDtypeStruct((E, DH), jnp.float32),
        grid=(E // BE,),
        in_specs=[
            pl.BlockSpec((BE, DE), lambda i: (i, 0)),
            pl.BlockSpec((DE, D), lambda i: (0, 0)),
            pl.BlockSpec((1, D), lambda i: (0, 0)),
        ],
        out_specs=pl.BlockSpec((BE, DH), lambda i: (i, 0)),
    )(edge_attr, w1ct, b1.reshape(1, D))


@functools.partial(
    pl.kernel,
    out_type=(
        jax.ShapeDtypeStruct((E, DH), jnp.float32),
        jax.ShapeDtypeStruct((NW, 2, D), jnp.float32),
        jax.ShapeDtypeStruct((NW, N), jnp.float32),
    ),
    mesh=_mesh,
    scratch_types=[
        pltpu.VMEM((NCHUNK, CH), jnp.int32),
        pltpu.VMEM((NCHUNK, CH), jnp.int32),
        pltpu.VMEM((2, CH, DH), jnp.float32),
        pltpu.VMEM((2, CH, DH), jnp.float32),
        pltpu.VMEM((2, CH, DH), jnp.float32),
        pltpu.VMEM((2, D), jnp.float32),
        pltpu.VMEM((N,), jnp.float32),
        pltpu.SemaphoreType.DMA,
        pltpu.SemaphoreType.DMA,
        pltpu.SemaphoreType.DMA,
        pltpu.SemaphoreType.DMA,
        pltpu.SemaphoreType.DMA,
        pltpu.SemaphoreType.DMA,
    ],
    compiler_params=_sc_params,
)
def _sc_pass1(xa_hbm, xb_hbm, ea_hbm, row_hbm, col_hbm,
              h_hbm, stats_hbm, cnt_hbm,
              rowsb, colsb, bufa, bufb, bufe, stats, counts,
              sa0, sa1, sb0, sb1, se0, se1):
    cid = lax.axis_index("c")
    sid = lax.axis_index("s")
    wid = sid * NC + cid
    base = wid * EPW
    sems_a = (sa0, sa1)
    sems_b = (sb0, sb1)
    sems_e = (se0, se1)

    zero16 = jnp.zeros((L,), jnp.float32)
    ones16 = jnp.full((L,), 1.0, jnp.float32)

    @pl.loop(0, N // L)
    def _(i):
        counts[pl.ds(i * L, L)] = zero16

    pltpu.sync_copy(row_hbm.at[wid], rowsb)
    pltpu.sync_copy(col_hbm.at[wid], colsb)

    def _issue(c, b):
        pltpu.async_copy(xa_hbm.at[rowsb.at[c]], bufa.at[b], sems_a[b])
        pltpu.async_copy(xb_hbm.at[colsb.at[c]], bufb.at[b], sems_b[b])
        pltpu.async_copy(ea_hbm.at[pl.ds(base + c * CH, CH)], bufe.at[b],
                         sems_e[b])

    def _wait(c, b):
        pltpu.make_async_copy(xa_hbm.at[rowsb.at[c]], bufa.at[b],
                              sems_a[b]).wait()
        pltpu.make_async_copy(xb_hbm.at[colsb.at[c]], bufb.at[b],
                              sems_b[b]).wait()
        pltpu.make_async_copy(ea_hbm.at[pl.ds(base + c * CH, CH)], bufe.at[b],
                              sems_e[b]).wait()

    def _compute(c, b):
        @pl.loop(0, CH // L)
        def _(j):
            idxv = rowsb[c, pl.ds(j * L, L)]
            plsc.addupdate_scatter(counts, [idxv], ones16)

        def _body(e, carry):
            out_s, out_q = [], []
            for k in range(G):
                sl = pl.ds(k * L, L)
                a = plsc.bitcast(bufa[b, e, sl], jnp.bfloat16)
                a2 = plsc.bitcast(bufb[b, e, sl], jnp.bfloat16)
                a3 = plsc.bitcast(bufe[b, e, sl], jnp.bfloat16)
                h = a + a2 + a3
                bufe[b, e, sl] = plsc.bitcast(h, jnp.float32)
                ha, hb = plsc.unpack(h, format=plsc.PackFormat.INTERLEAVED)
                out_s.append(carry[2 * k] + ha)
                out_s.append(carry[2 * k + 1] + hb)
                out_q.append(carry[2 * G + 2 * k] + ha * ha)
                out_q.append(carry[2 * G + 2 * k + 1] + hb * hb)
            return tuple(out_s + out_q)

        acc = lax.fori_loop(0, CH, _body, (zero16,) * (4 * G))
        for k in range(2 * G):
            plsc.addupdate(stats.at[0, pl.ds(k * L, L)], acc[k])
            plsc.addupdate(stats.at[1, pl.ds(k * L, L)], acc[2 * G + k])
        pltpu.sync_copy(bufe.at[b], h_hbm.at[pl.ds(base + c * CH, CH)])

    @pl.loop(0, 2 * G)
    def _(k):
        stats[0, pl.ds(k * L, L)] = zero16
        stats[1, pl.ds(k * L, L)] = zero16

    _issue(0, 0)

    @pl.loop(0, NCHUNK // 2)
    def _(t):
        for b in range(2):
            c = 2 * t + b
            _wait(c, b)
            _issue(c + 1, 1 - b)
            _compute(c, b)

    _wait(NCHUNK - 1, 0)
    _compute(NCHUNK - 1, 0)

    pltpu.sync_copy(stats, stats_hbm.at[wid])
    pltpu.sync_copy(counts, cnt_hbm.at[wid])


def _stats_body(stats_ref, g_ref, b_ref, ss_ref):
    s = jnp.sum(stats_ref[...], axis=0)
    mean = s[0:1, :] * (1.0 / E)
    ex2 = s[1:2, :] * (1.0 / E)
    var = ex2 - mean * mean
    inv = lax.rsqrt(var + 1e-5)
    scale = g_ref[...] * inv
    shift = b_ref[...] - mean * scale
    ss_ref[...] = jnp.concatenate([scale, shift], axis=0)


def _stats_reduce(stats, gamma_p, beta_p):
    return pl.pallas_call(
        _stats_body,
        out_shape=jax.ShapeDtypeStruct((2, D), jnp.float32),
    )(stats, gamma_p.reshape(1, D), beta_p.reshape(1, D))


@functools.partial(
    pl.kernel,
    out_type=jax.ShapeDtypeStruct((NC, N, D), jnp.float32),
    mesh=_mesh,
    scratch_types=[
        pltpu.VMEM((NCHUNK, CH), jnp.int32),
        pltpu.VMEM((2, CH, DH), jnp.float32),
        pltpu.VMEM((CH, D), jnp.float32),
        pltpu.VMEM((2, D), jnp.float32),
        pltpu.VMEM_SHARED((N, D), jnp.float32),
        pltpu.SemaphoreType.DMA,
        pltpu.SemaphoreType.DMA,
    ],
    compiler_params=_sc_params,
)
def _sc_pass2(h_hbm, row_hbm, ss_hbm, zeros_hbm,
              s_hbm,
              rowsb, buf, ybuf, ss, s_sh, sh0, sh1):
    cid = lax.axis_index("c")
    sid = lax.axis_index("s")
    wid = sid * NC + cid
    base = wid * EPW
    sems = (sh0, sh1)

    pltpu.sync_copy(ss_hbm, ss)
    pltpu.sync_copy(row_hbm.at[wid], rowsb)

    @pl.when(sid < NST)
    def _():
        pltpu.sync_copy(zeros_hbm, s_sh.at[pl.ds(sid * RPT, RPT)])

    sv = [plsc.pack(ss[0, pl.ds(2 * k * L, L)], ss[0, pl.ds((2 * k + 1) * L, L)],
                    format=plsc.PackFormat.INTERLEAVED) for k in range(G)]
    tv = [plsc.pack(ss[1, pl.ds(2 * k * L, L)], ss[1, pl.ds((2 * k + 1) * L, L)],
                    format=plsc.PackFormat.INTERLEAVED) for k in range(G)]

    def _issue(c, b):
        pltpu.async_copy(h_hbm.at[pl.ds(base + c * CH, CH)], buf.at[b],
                         sems[b])

    def _wait(c, b):
        pltpu.make_async_copy(h_hbm.at[pl.ds(base + c * CH, CH)], buf.at[b],
                              sems[b]).wait()

    def _compute(c, b):
        @pl.loop(0, CH)
        def _(e):
            for k in range(G):
                sl = pl.ds(k * L, L)
                h = plsc.bitcast(buf[b, e, sl], jnp.bfloat16)
                y = jnp.maximum(h * sv[k] + tv[k], jnp.bfloat16(0.0))
                ya, yb = plsc.unpack(y, format=plsc.PackFormat.INTERLEAVED)
                ybuf[e, pl.ds(2 * k * L, L)] = ya
                ybuf[e, pl.ds((2 * k + 1) * L, L)] = yb

        pltpu.sync_copy(ybuf, s_sh.at[rowsb.at[c]], add=True)

    _issue(0, 0)
    plsc.subcore_barrier()

    @pl.loop(0, NCHUNK // 2)
    def _(t):
        for b in range(2):
            c = 2 * t + b
            _wait(c, b)
            _issue(c + 1, 1 - b)
            _compute(c, b)

    _wait(NCHUNK - 1, 0)
    _compute(NCHUNK - 1, 0)

    plsc.subcore_barrier()

    @pl.when(sid < NST)
    def _():
        pltpu.sync_copy(
            s_sh.at[pl.ds(sid * RPT, RPT)],
            s_hbm.at[cid].at[pl.ds(sid * RPT, RPT)],
        )


def _final_body(s_ref, cnt_ref, x_ref, w_ref, b_ref, out_ref):
    s = s_ref[0] + s_ref[1]
    cnt = jnp.sum(cnt_ref[...], axis=0)[:, None]
    m = jnp.dot(s, w_ref[...], preferred_element_type=jnp.float32)
    out_ref[...] = (m + cnt * b_ref[...]) / (cnt + 1.0) + x_ref[...]


def _final(s_parts, cnts, x, w2tp, b2):
    return pl.pallas_call(
        _final_body,
        out_shape=jax.ShapeDtypeStruct((N, D), jnp.float32),
    )(s_parts, cnts, x, w2tp, b2.reshape(1, D))


def kernel(x, edge_index, edge_attr, W1, b1, gamma, beta, W2, b2):
    row = edge_index[0].astype(jnp.int32)
    col = edge_index[1].astype(jnp.int32)
    rows3 = row.reshape(NW, NCHUNK, CH)
    cols3 = col.reshape(NW, NCHUNK, CH)
    perm = jnp.asarray(_PERM, dtype=jnp.int32)
    w1abt = jnp.concatenate([W1[:, :D].T, W1[:, D : 2 * D].T], axis=1)
    w1ct = W1[:, 2 * D :].T
    w2tp = W2.T[perm, :]

    xa, xb = _node_proj(x, w1abt)
    ea = _edge_proj(edge_attr, w1ct, b1)
    h, stats, cnts = _sc_pass1(xa, xb, ea, rows3, cols3)
    ss = _stats_reduce(stats, gamma[perm], beta[perm])
    zeros = jnp.zeros((RPT, D), jnp.float32)
    s_parts = _sc_pass2(h, rows3, ss, zeros)
    return _final(s_parts, cnts, x, w2tp, b2)
